# Initial kernel scaffold; baseline (speedup 1.0000x reference)
import jax
import jax.numpy as jnp
from jax.experimental import pallas as pl


def kernel(x, edge_index, volume_id, node_params, edge_params):
    raise NotImplementedError("write your pallas kernel here")


# speedup vs baseline: 3.8626x; 3.8626x over previous
import functools

import jax
import jax.numpy as jnp
from jax import lax
from jax.experimental import pallas as pl
from jax.experimental.pallas import tpu as pltpu
from jax.experimental.pallas import tpu_sc as plsc

HIDDEN = 128
XDIM = 13
XPAD = 16
INP = 2 * XPAD
N_NODES = 10000
N_EDGES = 160000

NC, NS, L = 2, 16, 16
NW = NC * NS
EPAD = 163840
CHUNK = EPAD // NW
TILE = 1024
NT = CHUNK // TILE

EBLK = 256
NBLK = 400


def _ln(h, g, b):
    m = jnp.mean(h, axis=-1, keepdims=True)
    v = jnp.mean((h - m) * (h - m), axis=-1, keepdims=True)
    return (h - m) / jnp.sqrt(v + 1e-5) * g + b


def _sc_gather_body(xp_hbm, s_hbm, e_hbm, vid_hbm, ea_hbm, eb_hbm, c_hbm,
                    vid_v, sidx, eidx, ra, rb, cv, sem_a, sem_b):
    wid = lax.axis_index("s") * NC + lax.axis_index("c")
    pltpu.sync_copy(vid_hbm, vid_v)

    def tile_body(t, carry):
        base = wid * CHUNK + t * TILE
        pltpu.sync_copy(s_hbm.at[pl.ds(base, TILE)], sidx)
        pltpu.sync_copy(e_hbm.at[pl.ds(base, TILE)], eidx)
        handles = []
        for k in range(TILE // 128):
            sl = pl.ds(k * 128, 128)
            handles.append(pltpu.async_copy(xp_hbm.at[sidx.at[sl]], ra.at[sl], sem_a))
            handles.append(pltpu.async_copy(xp_hbm.at[eidx.at[sl]], rb.at[sl], sem_b))
        for h in handles:
            h.wait()
        one = jnp.full((L,), 1, jnp.int32)
        zero = jnp.full((L,), 0, jnp.int32)
        for k in range(TILE // L):
            sl = pl.ds(k * L, L)
            vs = plsc.load_gather(vid_v, [sidx[sl]])
            ve = plsc.load_gather(vid_v, [eidx[sl]])
            gs = jnp.where(vs >= 3, one, zero)
            ge = jnp.where(ve >= 3, one, zero)
            cv[sl] = 2 * gs + ge
        pltpu.sync_copy(ra, ea_hbm.at[pl.ds(base, TILE)])
        pltpu.sync_copy(rb, eb_hbm.at[pl.ds(base, TILE)])
        pltpu.sync_copy(cv, c_hbm.at[pl.ds(base, TILE)])
        return carry

    lax.fori_loop(0, NT, tile_body, 0)


def _sc_gather(xp, start, end, vid):
    mesh = plsc.VectorSubcoreMesh(core_axis_name="c", subcore_axis_name="s")
    f = functools.partial(
        pl.kernel,
        out_type=(
            jax.ShapeDtypeStruct((EPAD, XPAD), jnp.float32),
            jax.ShapeDtypeStruct((EPAD, XPAD), jnp.float32),
            jax.ShapeDtypeStruct((EPAD,), jnp.int32),
        ),
        mesh=mesh,
        scratch_types=[
            pltpu.VMEM((N_NODES,), jnp.int32),
            pltpu.VMEM((TILE,), jnp.int32),
            pltpu.VMEM((TILE,), jnp.int32),
            pltpu.VMEM((TILE, XPAD), jnp.float32),
            pltpu.VMEM((TILE, XPAD), jnp.float32),
            pltpu.VMEM((TILE,), jnp.int32),
            pltpu.SemaphoreType.DMA,
            pltpu.SemaphoreType.DMA,
        ],
        compiler_params=pltpu.CompilerParams(
            needs_layout_passes=False, use_tc_tiling_on_sc=False),
    )(_sc_gather_body)
    return f(xp, start, end, vid)


def _tc_edge_body(c_ref, a_ref, b_ref, w1_ref, b1_ref, g1_ref, be1_ref,
                  w2_ref, b2_ref, g2_ref, be2_ref, out_ref):
    inp = jnp.concatenate([a_ref[...], b_ref[...]], axis=1)
    c = c_ref[...]
    out = jnp.zeros((EBLK, HIDDEN), jnp.float32)
    for j, cval in ((0, 0), (1, 1), (2, 3)):
        h = jnp.dot(inp, w1_ref[j], preferred_element_type=jnp.float32) + b1_ref[j]
        h = jnp.maximum(_ln(h, g1_ref[j], be1_ref[j]), 0.0)
        h = jnp.dot(h, w2_ref[j], preferred_element_type=jnp.float32) + b2_ref[j]
        h = jnp.tanh(_ln(h, g2_ref[j], be2_ref[j]))
        out = jnp.where(c == cval, h, out)
    out_ref[...] = out


def _tc_edges(c2d, ea, eb, w1, b1, g1, be1, w2, b2, g2, be2):
    grid = EPAD // EBLK
    full = lambda s: pl.BlockSpec(s, lambda i: (0,) * len(s))
    return pl.pallas_call(
        _tc_edge_body,
        grid=(grid,),
        in_specs=[
            pl.BlockSpec((EBLK, 1), lambda i: (i, 0)),
            pl.BlockSpec((EBLK, XPAD), lambda i: (i, 0)),
            pl.BlockSpec((EBLK, XPAD), lambda i: (i, 0)),
            full((3, INP, HIDDEN)), full((3, HIDDEN)), full((3, HIDDEN)), full((3, HIDDEN)),
            full((3, HIDDEN, HIDDEN)), full((3, HIDDEN)), full((3, HIDDEN)), full((3, HIDDEN)),
        ],
        out_specs=pl.BlockSpec((EBLK, HIDDEN), lambda i: (i, 0)),
        out_shape=jax.ShapeDtypeStruct((EPAD, HIDDEN), jnp.float32),
    )(c2d, ea, eb, w1, b1, g1, be1, w2, b2, g2, be2)


def _tc_node_body(x_ref, vid_ref, w1_ref, b1_ref, g1_ref, be1_ref,
                  w2_ref, b2_ref, g2_ref, be2_ref, out_ref):
    xb = x_ref[...]
    vid = vid_ref[...]
    hs = []
    for j in range(2):
        h = jnp.dot(xb, w1_ref[j], preferred_element_type=jnp.float32) + b1_ref[j]
        h = jnp.maximum(_ln(h, g1_ref[j], be1_ref[j]), 0.0)
        h = jnp.dot(h, w2_ref[j], preferred_element_type=jnp.float32) + b2_ref[j]
        h = jnp.tanh(_ln(h, g2_ref[j], be2_ref[j]))
        hs.append(h)
    out_ref[...] = jnp.where(vid >= 3, hs[1], hs[0])


def _tc_nodes(xp, vid2d, w1, b1, g1, be1, w2, b2, g2, be2):
    grid = N_NODES // NBLK
    full = lambda s: pl.BlockSpec(s, lambda i: (0,) * len(s))
    return pl.pallas_call(
        _tc_node_body,
        grid=(grid,),
        in_specs=[
            pl.BlockSpec((NBLK, XPAD), lambda i: (i, 0)),
            pl.BlockSpec((NBLK, 1), lambda i: (i, 0)),
            full((2, XPAD, HIDDEN)), full((2, HIDDEN)), full((2, HIDDEN)), full((2, HIDDEN)),
            full((2, HIDDEN, HIDDEN)), full((2, HIDDEN)), full((2, HIDDEN)), full((2, HIDDEN)),
        ],
        out_specs=pl.BlockSpec((NBLK, HIDDEN), lambda i: (i, 0)),
        out_shape=jax.ShapeDtypeStruct((N_NODES, HIDDEN), jnp.float32),
    )(xp, vid2d, w1, b1, g1, be1, w2, b2, g2, be2)


def kernel(x, edge_index, volume_id, node_params, edge_params):
    xp = jnp.pad(x, ((0, 0), (0, XPAD - XDIM)))
    start = jnp.pad(edge_index[0], (0, EPAD - N_EDGES))
    end = jnp.pad(edge_index[1], (0, EPAD - N_EDGES))

    nw1 = jnp.stack([
        jnp.zeros((XPAD, HIDDEN), jnp.float32).at[0:3].set(p[0][0])
        for p in node_params
    ])
    nb1 = jnp.stack([p[0][1] for p in node_params])
    ng1 = jnp.stack([p[0][2] for p in node_params])
    nbe1 = jnp.stack([p[0][3] for p in node_params])
    nw2 = jnp.stack([p[1][0] for p in node_params])
    nb2 = jnp.stack([p[1][1] for p in node_params])
    ng2 = jnp.stack([p[1][2] for p in node_params])
    nbe2 = jnp.stack([p[1][3] for p in node_params])

    def pad_w1(w):
        wp = jnp.zeros((INP, HIDDEN), jnp.float32)
        wp = wp.at[0:XDIM].set(w[0:XDIM])
        wp = wp.at[XPAD:XPAD + XDIM].set(w[XDIM:2 * XDIM])
        return wp

    ew1 = jnp.stack([pad_w1(p[0][0]) for p in edge_params])
    eb1 = jnp.stack([p[0][1] for p in edge_params])
    eg1 = jnp.stack([p[0][2] for p in edge_params])
    ebe1 = jnp.stack([p[0][3] for p in edge_params])
    ew2 = jnp.stack([p[1][0] for p in edge_params])
    eb2 = jnp.stack([p[1][1] for p in edge_params])
    eg2 = jnp.stack([p[1][2] for p in edge_params])
    ebe2 = jnp.stack([p[1][3] for p in edge_params])

    ea, eb_rows, c = _sc_gather(xp, start, end, volume_id)
    c2d = c.reshape(EPAD, 1)

    encoded_edges = _tc_edges(c2d, ea, eb_rows, ew1, eb1, eg1, ebe1,
                              ew2, eb2, eg2, ebe2)[:N_EDGES]
    encoded_nodes = _tc_nodes(xp, volume_id.reshape(N_NODES, 1),
                              nw1, nb1, ng1, nbe1, nw2, nb2, ng2, nbe2)
    return (encoded_nodes, encoded_edges)


# speedup vs baseline: 4.7689x; 1.2346x over previous
import functools

import jax
import jax.numpy as jnp
from jax import lax
from jax.experimental import pallas as pl
from jax.experimental.pallas import tpu as pltpu
from jax.experimental.pallas import tpu_sc as plsc

HIDDEN = 128
XDIM = 13
XPAD = 16
INP = 2 * XPAD
N_NODES = 10000
N_EDGES = 160000

NC, NS, L = 2, 16, 16
NW = NC * NS
EPAD = 163840
CHUNK = EPAD // NW
TILE = 1024
NT = CHUNK // TILE

EBLK = 256
ESORT = EPAD + 4 * EBLK
NBLKS = ESORT // EBLK
BCPAD = 656
NBLK = 400

OTILE = 512
NOT = CHUNK // OTILE

_SC_PARAMS = pltpu.CompilerParams(
    needs_layout_passes=False, use_tc_tiling_on_sc=False)


def _ln(h, g, b):
    m = jnp.mean(h, axis=-1, keepdims=True)
    v = jnp.mean((h - m) * (h - m), axis=-1, keepdims=True)
    return (h - m) / jnp.sqrt(v + 1e-5) * g + b


def _wid():
    return lax.axis_index("s") * NC + lax.axis_index("c")


def _sc_combo_body(s_hbm, e_hbm, vid_hbm, c_hbm, hist_hbm,
                   vid_v, sidx, eidx, cv, hv):
    wid = _wid()
    pltpu.sync_copy(vid_hbm, vid_v)
    lanes = lax.broadcasted_iota(jnp.int32, (L,), 0)
    zero = jnp.zeros((L,), jnp.int32)
    one = jnp.full((L,), 1, jnp.int32)

    def tile(t, cnts):
        base = wid * CHUNK + t * TILE
        pltpu.sync_copy(s_hbm.at[pl.ds(base, TILE)], sidx)
        pltpu.sync_copy(e_hbm.at[pl.ds(base, TILE)], eidx)

        def vr(k, cn):
            sl = pl.ds(k * L, L)
            vs = plsc.load_gather(vid_v, [sidx[sl]])
            ve = plsc.load_gather(vid_v, [eidx[sl]])
            c = 2 * jnp.where(vs >= 3, one, zero) + jnp.where(ve >= 3, one, zero)
            s = jnp.where(c == 3, 2 * one, jnp.where(c == 2, 3 * one, c))
            cv[sl] = s
            return (cn[0] + plsc.all_reduce_population_count(s == 0),
                    cn[1] + plsc.all_reduce_population_count(s == 1),
                    cn[2] + plsc.all_reduce_population_count(s == 2),
                    cn[3] + plsc.all_reduce_population_count(s == 3))

        cnts = lax.fori_loop(0, TILE // L, vr, cnts)
        pltpu.sync_copy(cv, c_hbm.at[pl.ds(base, TILE)])
        return cnts

    c0, c1, c2, c3 = lax.fori_loop(0, NT, tile, (zero, zero, zero, zero))
    comb = jnp.where(lanes == 0, c0,
           jnp.where(lanes == 1, c1,
           jnp.where(lanes == 2, c2,
           jnp.where(lanes == 3, c3, zero))))
    hv[...] = comb
    pltpu.sync_copy(hv, hist_hbm.at[wid])


def _sc_combo(start, end, vid):
    mesh = plsc.VectorSubcoreMesh(core_axis_name="c", subcore_axis_name="s")
    f = functools.partial(
        pl.kernel,
        out_type=(
            jax.ShapeDtypeStruct((EPAD,), jnp.int32),
            jax.ShapeDtypeStruct((NW, L), jnp.int32),
        ),
        mesh=mesh,
        scratch_types=[
            pltpu.VMEM((N_NODES,), jnp.int32),
            pltpu.VMEM((TILE,), jnp.int32),
            pltpu.VMEM((TILE,), jnp.int32),
            pltpu.VMEM((TILE,), jnp.int32),
            pltpu.VMEM((L,), jnp.int32),
        ],
        compiler_params=_SC_PARAMS,
    )(_sc_combo_body)
    return f(start, end, vid)


def _seg_bases(hist_hbm, histv, csv, wid):
    pltpu.sync_copy(hist_hbm, histv)
    lanes = lax.broadcasted_iota(jnp.int32, (L,), 0)
    widv = jnp.full((L,), wid, jnp.int32)
    totals, bases = [], []
    for j in range(4):
        jv = jnp.full((L,), j, jnp.int32)
        h_lo = plsc.load_gather(histv, [lanes, jv])
        h_hi = plsc.load_gather(histv, [lanes + 16, jv])
        cs_lo = plsc.cumsum(h_lo)
        csv[pl.ds(0, L)] = cs_lo
        tot_lo_v = plsc.load_gather(csv, [jnp.full((L,), 15, jnp.int32)])
        cs_hi = plsc.cumsum(h_hi) + tot_lo_v
        csv[pl.ds(L, L)] = cs_hi
        tot_v = plsc.load_gather(csv, [jnp.full((L,), 31, jnp.int32)])
        my_incl = plsc.load_gather(csv, [widv])
        my_cnt = plsc.load_gather(histv, [widv, jv])
        totals.append(tot_v)
        bases.append(my_incl - my_cnt)
    align = lambda v: (v + (EBLK - 1)) & (-EBLK)
    a = [jnp.zeros((L,), jnp.int32)]
    for j in range(3):
        a.append(align(a[j] + totals[j]))
    seg_base = [bases[j] + a[j] for j in range(4)]
    return a, seg_base


def _sc_route_body(xp_hbm, s_hbm, e_hbm, c_hbm, hist_hbm,
                   sortA_hbm, sortB_hbm, dst_hbm, blk_hbm,
                   histv, csv, sidx, eidx, sv, dst3, ra, rb, bv,
                   sem_g, sem_s):
    wid = _wid()
    a, seg = _seg_bases(hist_hbm, histv, csv, wid)
    lanes = lax.broadcasted_iota(jnp.int32, (L,), 0)
    one = jnp.full((L,), 1, jnp.int32)

    @pl.when(wid == 0)
    def _():
        for m in range(BCPAD // L):
            row = (lanes + m * L) * EBLK
            v = jnp.where(row < a[1], 0 * one,
                jnp.where(row < a[2], one,
                jnp.where(row < a[3], 2 * one, 3 * one)))
            bv[pl.ds(m * L, L)] = v
        pltpu.sync_copy(bv, blk_hbm)

    def tile(t, seg_c):
        base = wid * CHUNK + t * TILE
        pltpu.sync_copy(s_hbm.at[pl.ds(base, TILE)], sidx)
        pltpu.sync_copy(e_hbm.at[pl.ds(base, TILE)], eidx)
        pltpu.sync_copy(c_hbm.at[pl.ds(base, TILE)], sv)

        for jj in range(TILE // 128):
            def vr(k, sc):
                b0, b1, b2, b3 = sc
                sl = pl.ds(jj * 128 + k * L, L)
                s = sv[sl]
                m0, m1, m2, m3 = (s == 0), (s == 1), (s == 2), (s == 3)
                d0 = b0 + plsc.cumsum(one, mask=m0) - 1
                d1 = b1 + plsc.cumsum(one, mask=m1) - 1
                d2 = b2 + plsc.cumsum(one, mask=m2) - 1
                d3 = b3 + plsc.cumsum(one, mask=m3) - 1
                dst = jnp.where(m0, d0, jnp.where(m1, d1, jnp.where(m2, d2, d3)))
                dst3[jj, 0, pl.ds(k * L, L)] = dst
                return (b0 + plsc.all_reduce_population_count(m0),
                        b1 + plsc.all_reduce_population_count(m1),
                        b2 + plsc.all_reduce_population_count(m2),
                        b3 + plsc.all_reduce_population_count(m3))

            seg_c = lax.fori_loop(0, 128 // L, vr, seg_c)

        handles = []
        for jj in range(TILE // 128):
            sl = pl.ds(jj * 128, 128)
            handles.append(pltpu.async_copy(xp_hbm.at[sidx.at[sl]], ra.at[sl], sem_g))
            handles.append(pltpu.async_copy(xp_hbm.at[eidx.at[sl]], rb.at[sl], sem_g))
        for h in handles:
            h.wait()
        handles = []
        for jj in range(TILE // 128):
            sl = pl.ds(jj * 128, 128)
            handles.append(pltpu.async_copy(ra.at[sl], sortA_hbm.at[dst3.at[jj, 0]], sem_s))
            handles.append(pltpu.async_copy(rb.at[sl], sortB_hbm.at[dst3.at[jj, 0]], sem_s))
        pltpu.sync_copy(dst3, dst_hbm.at[pl.ds(wid * (CHUNK // 128) + t * (TILE // 128), TILE // 128)])
        for h in handles:
            h.wait()
        return seg_c

    lax.fori_loop(0, NT, tile, tuple(seg))


def _sc_route(xp, start, end, codes, hist):
    mesh = plsc.VectorSubcoreMesh(core_axis_name="c", subcore_axis_name="s")
    f = functools.partial(
        pl.kernel,
        out_type=(
            jax.ShapeDtypeStruct((ESORT, XPAD), jnp.float32),
            jax.ShapeDtypeStruct((ESORT, XPAD), jnp.float32),
            jax.ShapeDtypeStruct((EPAD // 128, 1, 128), jnp.int32),
            jax.ShapeDtypeStruct((BCPAD,), jnp.int32),
        ),
        mesh=mesh,
        scratch_types=[
            pltpu.VMEM((NW, L), jnp.int32),
            pltpu.VMEM((2 * L,), jnp.int32),
            pltpu.VMEM((TILE,), jnp.int32),
            pltpu.VMEM((TILE,), jnp.int32),
            pltpu.VMEM((TILE,), jnp.int32),
            pltpu.VMEM((TILE // 128, 1, 128), jnp.int32),
            pltpu.VMEM((TILE, XPAD), jnp.float32),
            pltpu.VMEM((TILE, XPAD), jnp.float32),
            pltpu.VMEM((BCPAD,), jnp.int32),
            pltpu.SemaphoreType.DMA,
            pltpu.SemaphoreType.DMA,
        ],
        compiler_params=_SC_PARAMS,
    )(_sc_route_body)
    return f(xp, start, end, codes, hist)


def _sc_unsort_body(outs_hbm, dst_hbm, enc_hbm, dstv, rows, sem):
    wid = _wid()

    def tile(t, carry):
        base = wid * CHUNK + t * OTILE
        pltpu.sync_copy(dst_hbm.at[pl.ds(base // 128, OTILE // 128)], dstv)
        handles = []
        for j in range(OTILE // 128):
            handles.append(pltpu.async_copy(
                outs_hbm.at[dstv.at[j, 0]], rows.at[pl.ds(j * 128, 128)], sem))
        for h in handles:
            h.wait()
        pltpu.sync_copy(rows, enc_hbm.at[pl.ds(base, OTILE)])
        return carry

    lax.fori_loop(0, NOT, tile, 0)


def _sc_unsort(out_sorted, dst):
    mesh = plsc.VectorSubcoreMesh(core_axis_name="c", subcore_axis_name="s")
    f = functools.partial(
        pl.kernel,
        out_type=jax.ShapeDtypeStruct((EPAD, HIDDEN), jnp.float32),
        mesh=mesh,
        scratch_types=[
            pltpu.VMEM((OTILE // 128, 1, 128), jnp.int32),
            pltpu.VMEM((OTILE, HIDDEN), jnp.float32),
            pltpu.SemaphoreType.DMA,
        ],
        compiler_params=_SC_PARAMS,
    )(_sc_unsort_body)
    return f(out_sorted, dst)


def _tc_edge_body(cb_ref, a_ref, b_ref, w1_ref, b1_ref, g1_ref, be1_ref,
                  w2_ref, b2_ref, g2_ref, be2_ref, out_ref):
    inp = jnp.concatenate([a_ref[...], b_ref[...]], axis=1)
    h = jnp.dot(inp, w1_ref[0], preferred_element_type=jnp.float32) + b1_ref[0]
    h = jnp.maximum(_ln(h, g1_ref[0], be1_ref[0]), 0.0)
    h = jnp.dot(h, w2_ref[0], preferred_element_type=jnp.float32) + b2_ref[0]
    h = jnp.tanh(_ln(h, g2_ref[0], be2_ref[0]))
    seg = cb_ref[pl.program_id(0)]
    out_ref[...] = jnp.where(seg == 3, jnp.zeros_like(h), h)


def _tc_edges(blkseg, ea, eb, w1, b1, g1, be1, w2, b2, g2, be2):
    grid_spec = pltpu.PrefetchScalarGridSpec(
        num_scalar_prefetch=1,
        grid=(NBLKS,),
        in_specs=[
            pl.BlockSpec((EBLK, XPAD), lambda i, cb: (i, 0)),
            pl.BlockSpec((EBLK, XPAD), lambda i, cb: (i, 0)),
            pl.BlockSpec((1, INP, HIDDEN), lambda i, cb: (cb[i], 0, 0)),
            pl.BlockSpec((1, 1, HIDDEN), lambda i, cb: (cb[i], 0, 0)),
            pl.BlockSpec((1, 1, HIDDEN), lambda i, cb: (cb[i], 0, 0)),
            pl.BlockSpec((1, 1, HIDDEN), lambda i, cb: (cb[i], 0, 0)),
            pl.BlockSpec((1, HIDDEN, HIDDEN), lambda i, cb: (cb[i], 0, 0)),
            pl.BlockSpec((1, 1, HIDDEN), lambda i, cb: (cb[i], 0, 0)),
            pl.BlockSpec((1, 1, HIDDEN), lambda i, cb: (cb[i], 0, 0)),
            pl.BlockSpec((1, 1, HIDDEN), lambda i, cb: (cb[i], 0, 0)),
        ],
        out_specs=pl.BlockSpec((EBLK, HIDDEN), lambda i, cb: (i, 0)),
    )
    return pl.pallas_call(
        _tc_edge_body,
        grid_spec=grid_spec,
        out_shape=jax.ShapeDtypeStruct((ESORT, HIDDEN), jnp.float32),
    )(blkseg, ea, eb, w1, b1, g1, be1, w2, b2, g2, be2)


def _tc_node_body(x_ref, vid_ref, w1_ref, b1_ref, g1_ref, be1_ref,
                  w2_ref, b2_ref, g2_ref, be2_ref, out_ref):
    xb = x_ref[...]
    vid = vid_ref[...]
    hs = []
    for j in range(2):
        h = jnp.dot(xb, w1_ref[j], preferred_element_type=jnp.float32) + b1_ref[j]
        h = jnp.maximum(_ln(h, g1_ref[j], be1_ref[j]), 0.0)
        h = jnp.dot(h, w2_ref[j], preferred_element_type=jnp.float32) + b2_ref[j]
        h = jnp.tanh(_ln(h, g2_ref[j], be2_ref[j]))
        hs.append(h)
    out_ref[...] = jnp.where(vid >= 3, hs[1], hs[0])


def _tc_nodes(xp, vid2d, w1, b1, g1, be1, w2, b2, g2, be2):
    grid = N_NODES // NBLK
    full = lambda s: pl.BlockSpec(s, lambda i: (0,) * len(s))
    return pl.pallas_call(
        _tc_node_body,
        grid=(grid,),
        in_specs=[
            pl.BlockSpec((NBLK, XPAD), lambda i: (i, 0)),
            pl.BlockSpec((NBLK, 1), lambda i: (i, 0)),
            full((2, XPAD, HIDDEN)), full((2, HIDDEN)), full((2, HIDDEN)), full((2, HIDDEN)),
            full((2, HIDDEN, HIDDEN)), full((2, HIDDEN)), full((2, HIDDEN)), full((2, HIDDEN)),
        ],
        out_specs=pl.BlockSpec((NBLK, HIDDEN), lambda i: (i, 0)),
        out_shape=jax.ShapeDtypeStruct((N_NODES, HIDDEN), jnp.float32),
    )(xp, vid2d, w1, b1, g1, be1, w2, b2, g2, be2)


def kernel(x, edge_index, volume_id, node_params, edge_params):
    xp = jnp.pad(x, ((0, 0), (0, XPAD - XDIM)))
    start = jnp.pad(edge_index[0], (0, EPAD - N_EDGES))
    end = jnp.pad(edge_index[1], (0, EPAD - N_EDGES))

    nw1 = jnp.stack([
        jnp.zeros((XPAD, HIDDEN), jnp.float32).at[0:3].set(p[0][0])
        for p in node_params
    ])
    nb1 = jnp.stack([p[0][1] for p in node_params])
    ng1 = jnp.stack([p[0][2] for p in node_params])
    nbe1 = jnp.stack([p[0][3] for p in node_params])
    nw2 = jnp.stack([p[1][0] for p in node_params])
    nb2 = jnp.stack([p[1][1] for p in node_params])
    ng2 = jnp.stack([p[1][2] for p in node_params])
    nbe2 = jnp.stack([p[1][3] for p in node_params])

    def pad_w1(w):
        wp = jnp.zeros((INP, HIDDEN), jnp.float32)
        wp = wp.at[0:XDIM].set(w[0:XDIM])
        wp = wp.at[XPAD:XPAD + XDIM].set(w[XDIM:2 * XDIM])
        return wp

    zv = jnp.zeros((HIDDEN,), jnp.float32)
    ew1 = jnp.stack([pad_w1(p[0][0]) for p in edge_params]
                    + [jnp.zeros((INP, HIDDEN), jnp.float32)])
    eb1 = jnp.stack([p[0][1] for p in edge_params] + [zv])
    eg1 = jnp.stack([p[0][2] for p in edge_params] + [zv])
    ebe1 = jnp.stack([p[0][3] for p in edge_params] + [zv])
    ew2 = jnp.stack([p[1][0] for p in edge_params]
                    + [jnp.zeros((HIDDEN, HIDDEN), jnp.float32)])
    eb2 = jnp.stack([p[1][1] for p in edge_params] + [zv])
    eg2 = jnp.stack([p[1][2] for p in edge_params] + [zv])
    ebe2 = jnp.stack([p[1][3] for p in edge_params] + [zv])

    codes, hist = _sc_combo(start, end, volume_id)
    sortA, sortB, dst, blkseg = _sc_route(xp, start, end, codes, hist)
    r3 = lambda v: v.reshape(4, 1, HIDDEN)
    out_sorted = _tc_edges(blkseg[:NBLKS], sortA, sortB,
                           ew1, r3(eb1), r3(eg1), r3(ebe1),
                           ew2, r3(eb2), r3(eg2), r3(ebe2))
    encoded_edges = _sc_unsort(out_sorted, dst)[:N_EDGES]
    encoded_nodes = _tc_nodes(xp, volume_id.reshape(N_NODES, 1),
                              nw1, nb1, ng1, nbe1, nw2, nb2, ng2, nbe2)
    return (encoded_nodes, encoded_edges)
